# baseline (speedup 1.0000x reference)
import functools

import jax
import jax.numpy as jnp
from jax import lax
from jax.experimental import pallas as pl
from jax.experimental.pallas import tpu as pltpu
from jax.experimental.pallas import tpu_sc as plsc

N = 10000
E = 160000
D = 256
M = 4096
H = D // 2

BN = 400
NBLK = N // BN

NTILE = 16
EPT = E // NTILE
B = 64
NCH = 5
CB = 32
CHW = CB * B
TB = 28
TSZ = 2 * B
ACCROWS = N + 8
RPT_A = 632
RPT_B = N - 15 * RPT_A

_mesh = plsc.VectorSubcoreMesh(core_axis_name="c", subcore_axis_name="s")


def _make_hops(nhops):
  @functools.partial(
    pl.kernel,
    out_type=jax.ShapeDtypeStruct((nhops, 2, N, H), jnp.float32),
    mesh=_mesh,
    scratch_types=[
        pltpu.VMEM((CHW,), jnp.int32),
        pltpu.VMEM((CHW,), jnp.int32),
        pltpu.VMEM((CHW,), jnp.int32),
        pltpu.VMEM((CHW,), jnp.int32),
        pltpu.VMEM((B, H), jnp.float32),
        pltpu.VMEM((B, H), jnp.float32),
        pltpu.VMEM((B, H), jnp.float32),
        pltpu.VMEM((B, H), jnp.float32),
        pltpu.SemaphoreType.DMA,
        pltpu.SemaphoreType.DMA,
        pltpu.SemaphoreType.DMA,
        pltpu.SemaphoreType.DMA,
        pltpu.SemaphoreType.DMA,
        pltpu.SemaphoreType.DMA,
        pltpu.SemaphoreType.DMA,
        pltpu.SemaphoreType.DMA,
        pltpu.SemaphoreType.DMA,
        pltpu.SemaphoreType.DMA,
        pltpu.VMEM_SHARED((ACCROWS, H), jnp.float32),
    ],
  )
  def _hops(z_hbm, src_hbm, dst_hbm, tsrc_hbm, tdst_hbm, zero_hbm, out_hbm,
            src_a, dst_a, src_b, dst_b, b0, b1, b2, b3,
            g0, g1, g2, g3, s0, s1, s2, s3, si0, si1, acc):
    c = lax.axis_index("c")
    w = lax.axis_index("s")
    bufs = (b0, b1, b2, b3)
    gsems = (g0, g1, g2, g3)
    ssems = (s0, s1, s2, s3)

    dummy = zero_hbm.at[pl.ds(0, B)]

    def drain(sem, buf):
        pltpu.make_async_copy(dummy, buf, sem).wait()

    def zero_stripe():
        @pl.when(w < 15)
        def _():
            pltpu.sync_copy(zero_hbm,
                            acc.at[pl.ds(w * RPT_A, RPT_A)])

        @pl.when(w == 15)
        def _():
            pltpu.sync_copy(zero_hbm.at[pl.ds(0, RPT_B)],
                            acc.at[pl.ds(15 * RPT_A, RPT_B)])

    zero_stripe()
    plsc.subcore_barrier()

    for hop in range(nhops):
        zc = z_hbm.at[c] if hop == 0 else out_hbm.at[hop - 1].at[c]

        pltpu.sync_copy(src_hbm.at[w].at[pl.ds(0, CHW)], src_a)
        pltpu.sync_copy(dst_hbm.at[w].at[pl.ds(0, CHW)], dst_a)

        for ch in range(NCH):
            src_c, dst_c = (src_a, dst_a) if ch % 2 == 0 else (src_b, dst_b)
            src_n, dst_n = (src_b, dst_b) if ch % 2 == 0 else (src_a, dst_a)
            nxtw = CHW if ch + 1 < NCH - 1 else TB * B
            if ch + 1 < NCH:
                pltpu.async_copy(
                    src_hbm.at[w].at[pl.ds((ch + 1) * CHW, nxtw)],
                    src_n.at[pl.ds(0, nxtw)], si0)
                pltpu.async_copy(
                    dst_hbm.at[w].at[pl.ds((ch + 1) * CHW, nxtw)],
                    dst_n.at[pl.ds(0, nxtw)], si1)

            nb = CB if ch < NCH - 1 else TB

            if ch > 0:
                for u in range(4):
                    drain(ssems[u], bufs[u])
            for u in range(3):
                pltpu.async_copy(zc.at[src_c.at[pl.ds(u * B, B)]],
                                 bufs[u], gsems[u])

            def quad(k, carry):
                for u in range(4):
                    j = 4 * k + u
                    drain(gsems[u], bufs[u])
                    pltpu.async_copy(bufs[u],
                                     acc.at[dst_c.at[pl.ds(j * B, B)]],
                                     ssems[u], add=True)
                    v = (u + 3) % 4

                    @pl.when(j + 3 < nb)
                    def _():
                        @pl.when(j >= 1)
                        def _():
                            drain(ssems[v], bufs[v])
                        pltpu.async_copy(
                            zc.at[src_c.at[pl.ds((j + 3) * B, B)]],
                            bufs[v], gsems[v])
                return carry

            lax.fori_loop(0, nb // 4, quad, 0)

            if ch == NCH - 1:
                pltpu.sync_copy(tsrc_hbm.at[w], src_b.at[pl.ds(0, TSZ)])
                pltpu.sync_copy(tdst_hbm.at[w], dst_b.at[pl.ds(0, TSZ)])
                for t in range(2):
                    drain(ssems[t], bufs[t])
                    pltpu.async_copy(
                        zc.at[src_b.at[pl.ds(t * B, B)]], bufs[t], gsems[t])
                    drain(gsems[t], bufs[t])
                    pltpu.async_copy(bufs[t],
                                     acc.at[dst_b.at[pl.ds(t * B, B)]],
                                     ssems[t], add=True)

            if ch + 1 < NCH:
                pltpu.make_async_copy(
                    src_hbm.at[w].at[pl.ds((ch + 1) * CHW, nxtw)],
                    src_n.at[pl.ds(0, nxtw)], si0).wait()
                pltpu.make_async_copy(
                    dst_hbm.at[w].at[pl.ds((ch + 1) * CHW, nxtw)],
                    dst_n.at[pl.ds(0, nxtw)], si1).wait()

        for u in range(4):
            drain(ssems[u], bufs[u])

        plsc.subcore_barrier()

        @pl.when(w < 15)
        def _():
            pltpu.sync_copy(acc.at[pl.ds(w * RPT_A, RPT_A)],
                            out_hbm.at[hop].at[c].at[pl.ds(w * RPT_A, RPT_A)])

        @pl.when(w == 15)
        def _():
            pltpu.sync_copy(acc.at[pl.ds(15 * RPT_A, RPT_B)],
                            out_hbm.at[hop].at[c].at[pl.ds(15 * RPT_A, RPT_B)])

        if hop < nhops - 1:
            zero_stripe()
            plsc.subcore_barrier()

  return _hops


_hops_ab = (_make_hops(2), _make_hops(2))


def _sc_hops(feat_a, edge_index):
    srcp = edge_index[0].reshape(NTILE, EPT)
    dstp = edge_index[1].reshape(NTILE, EPT)
    ntail = EPT - 4 * CHW - TB * B
    tsrc = jnp.concatenate(
        [srcp[:, EPT - ntail:],
         jnp.zeros((NTILE, TSZ - ntail), jnp.int32)], axis=1)
    tdst = jnp.concatenate(
        [dstp[:, EPT - ntail:],
         jnp.full((NTILE, TSZ - ntail), N, jnp.int32)], axis=1)
    zeros = jnp.zeros((RPT_A, H), jnp.float32)

    z = feat_a.reshape(N, 2, H).transpose(1, 0, 2)
    zs_a = _hops_ab[0](z, srcp, dstp, tsrc, tdst, zeros)
    zs_b = _hops_ab[1](zs_a[1], srcp, dstp, tsrc, tdst, zeros)
    return zs_a, zs_b


def _part_body(feat_a_ref, pm_ref, feat_b_ref, w_fuse_t_ref, w_prev_t_ref,
               bias_ref, part_ref, fw_ref):
    i = pl.program_id(0)

    @pl.when(i == 0)
    def _():
        fw_ref[...] = jnp.dot(feat_b_ref[...], w_fuse_t_ref[...],
                              preferred_element_type=jnp.float32)

    acc = jnp.dot(feat_a_ref[...], w_prev_t_ref[...],
                  preferred_element_type=jnp.float32)
    acc += jnp.dot(pm_ref[...], fw_ref[...],
                   preferred_element_type=jnp.float32)
    part_ref[...] = acc + bias_ref[...]


def _mid_body(part_ref, z1l_ref, z1r_ref, z2l_ref, z2r_ref,
              w1l_ref, w1r_ref, w2l_ref, w2r_ref, out_ref):
    acc = part_ref[...]
    acc += jnp.dot(z1l_ref[0, 0], w1l_ref[...],
                   preferred_element_type=jnp.float32)
    acc += jnp.dot(z1r_ref[0, 0], w1r_ref[...],
                   preferred_element_type=jnp.float32)
    acc += jnp.dot(z2l_ref[0, 0], w2l_ref[...],
                   preferred_element_type=jnp.float32)
    acc += jnp.dot(z2r_ref[0, 0], w2r_ref[...],
                   preferred_element_type=jnp.float32)
    out_ref[...] = acc


def _main_body(part2_ref, z4l_ref, z4r_ref, w3l_ref, w3r_ref,
               r_ref, psum_ref, psq_ref):
    acc = part2_ref[...]
    acc += jnp.dot(z4l_ref[0, 0], w3l_ref[...],
                   preferred_element_type=jnp.float32)
    acc += jnp.dot(z4r_ref[0, 0], w3r_ref[...],
                   preferred_element_type=jnp.float32)

    col = jax.lax.broadcasted_iota(jnp.int32, (BN, D), 1)
    acc = jnp.where(col >= D // 2, jnp.maximum(acc, 0.0), acc)

    r_ref[...] = acc
    psum_ref[...] = jnp.broadcast_to(jnp.sum(acc, axis=0, keepdims=True) / 8.0,
                                     (8, D))
    psq_ref[...] = jnp.broadcast_to(jnp.sum(acc * acc, axis=0, keepdims=True) / 8.0,
                                    (8, D))


def _bn_body(r_ref, psum_ref, psq_ref, gamma_ref, beta_ref, out_ref):
    mean = jnp.sum(psum_ref[...], axis=0, keepdims=True) / N
    var = jnp.sum(psq_ref[...], axis=0, keepdims=True) / N - mean * mean
    scale = jax.lax.rsqrt(var + 1e-5) * gamma_ref[...]
    out_ref[...] = (r_ref[...] - mean) * scale + beta_ref[...]


def _dense_part(zs_a, zs_b, part, W_rad, bn_gamma, bn_beta):
    w1t = W_rad[0].T
    w2t = W_rad[1].T
    w3t = W_rad[2].T

    zspec_a = [
        pl.BlockSpec((1, 1, BN, H), lambda i, k=k, cc=cc: (k, cc, i, 0))
        for (k, cc) in ((0, 0), (0, 1), (1, 0), (1, 1))
    ]
    part2 = pl.pallas_call(
        _mid_body,
        grid=(NBLK,),
        in_specs=[pl.BlockSpec((BN, D), lambda i: (i, 0))] + zspec_a + [
            pl.BlockSpec((H, D), lambda i: (0, 0)),
            pl.BlockSpec((H, D), lambda i: (1, 0)),
            pl.BlockSpec((H, D), lambda i: (0, 0)),
            pl.BlockSpec((H, D), lambda i: (1, 0)),
        ],
        out_specs=pl.BlockSpec((BN, D), lambda i: (i, 0)),
        out_shape=jax.ShapeDtypeStruct((N, D), jnp.float32),
    )(part, zs_a, zs_a, zs_a, zs_a, w1t, w1t, w2t, w2t)

    zspec_b = [
        pl.BlockSpec((1, 1, BN, H), lambda i, cc=cc: (1, cc, i, 0))
        for cc in (0, 1)
    ]
    r, psum, psq = pl.pallas_call(
        _main_body,
        grid=(NBLK,),
        in_specs=[pl.BlockSpec((BN, D), lambda i: (i, 0))] + zspec_b + [
            pl.BlockSpec((H, D), lambda i: (0, 0)),
            pl.BlockSpec((H, D), lambda i: (1, 0)),
        ],
        out_specs=[
            pl.BlockSpec((BN, D), lambda i: (i, 0)),
            pl.BlockSpec((8, D), lambda i: (i, 0)),
            pl.BlockSpec((8, D), lambda i: (i, 0)),
        ],
        out_shape=[
            jax.ShapeDtypeStruct((N, D), jnp.float32),
            jax.ShapeDtypeStruct((NBLK * 8, D), jnp.float32),
            jax.ShapeDtypeStruct((NBLK * 8, D), jnp.float32),
        ],
    )(part2, zs_b, zs_b, w3t, w3t)

    out = pl.pallas_call(
        _bn_body,
        grid=(NBLK,),
        in_specs=[
            pl.BlockSpec((BN, D), lambda i: (i, 0)),
            pl.BlockSpec((NBLK * 8, D), lambda i: (0, 0)),
            pl.BlockSpec((NBLK * 8, D), lambda i: (0, 0)),
            pl.BlockSpec((1, D), lambda i: (0, 0)),
            pl.BlockSpec((1, D), lambda i: (0, 0)),
        ],
        out_specs=pl.BlockSpec((BN, D), lambda i: (i, 0)),
        out_shape=jax.ShapeDtypeStruct((N, D), jnp.float32),
    )(r, psum, psq, bn_gamma.reshape(1, D), bn_beta.reshape(1, D))
    return out


def kernel(feat_a, feat_b, deg, pm_pd, edge_index,
           W_prev, b_prev, W_deg, b_deg, W_rad, b_rad,
           W_fuse, b_fuse, bn_gamma, bn_beta):
    bias = b_prev + b_rad[0] + b_rad[1] + b_rad[2] + b_fuse

    part = pl.pallas_call(
        _part_body,
        grid=(NBLK,),
        in_specs=[
            pl.BlockSpec((BN, D), lambda i: (i, 0)),
            pl.BlockSpec((BN, M), lambda i: (i, 0)),
            pl.BlockSpec((M, D), lambda i: (0, 0)),
            pl.BlockSpec((D, D), lambda i: (0, 0)),
            pl.BlockSpec((D, D), lambda i: (0, 0)),
            pl.BlockSpec((1, D), lambda i: (0, 0)),
        ],
        out_specs=pl.BlockSpec((BN, D), lambda i: (i, 0)),
        out_shape=jax.ShapeDtypeStruct((N, D), jnp.float32),
        scratch_shapes=[pltpu.VMEM((M, D), jnp.float32)],
    )(feat_a, pm_pd, feat_b, W_fuse.T, W_prev.T, bias.reshape(1, D))

    zs_a, zs_b = _sc_hops(feat_a, edge_index)
    return _dense_part(zs_a, zs_b, part, W_rad, bn_gamma, bn_beta)


# speedup vs baseline: 1.0120x; 1.0120x over previous
import functools

import jax
import jax.numpy as jnp
from jax import lax
from jax.experimental import pallas as pl
from jax.experimental.pallas import tpu as pltpu
from jax.experimental.pallas import tpu_sc as plsc

N = 10000
E = 160000
D = 256
M = 4096
H = D // 2

BN = 400
NBLK = N // BN

NTILE = 16
EPT = E // NTILE
B = 64
NCH = 5
CB = 32
CHW = CB * B
TB = 28
TSZ = 2 * B
ACCROWS = N + 8
RPT_A = 632
RPT_B = N - 15 * RPT_A

_mesh = plsc.VectorSubcoreMesh(core_axis_name="c", subcore_axis_name="s")


@functools.partial(
    pl.kernel,
    out_type=jax.ShapeDtypeStruct((4, 2, N, H), jnp.float32),
    mesh=_mesh,
    scratch_types=[
        pltpu.VMEM((CHW,), jnp.int32),
        pltpu.VMEM((CHW,), jnp.int32),
        pltpu.VMEM((CHW,), jnp.int32),
        pltpu.VMEM((CHW,), jnp.int32),
        pltpu.VMEM((B, H), jnp.float32),
        pltpu.VMEM((B, H), jnp.float32),
        pltpu.VMEM((B, H), jnp.float32),
        pltpu.VMEM((B, H), jnp.float32),
        pltpu.SemaphoreType.DMA,
        pltpu.SemaphoreType.DMA,
        pltpu.SemaphoreType.DMA,
        pltpu.SemaphoreType.DMA,
        pltpu.SemaphoreType.DMA,
        pltpu.SemaphoreType.DMA,
        pltpu.SemaphoreType.DMA,
        pltpu.SemaphoreType.DMA,
        pltpu.SemaphoreType.DMA,
        pltpu.SemaphoreType.DMA,
        pltpu.VMEM_SHARED((ACCROWS, H), jnp.float32),
    ],
)
def _hops4(z_hbm, src_hbm, dst_hbm, tsrc_hbm, tdst_hbm, zero_hbm, out_hbm,
           src_a, dst_a, src_b, dst_b, b0, b1, b2, b3,
           g0, g1, g2, g3, s0, s1, s2, s3, si0, si1, acc):
    c = lax.axis_index("c")
    w = lax.axis_index("s")
    bufs = (b0, b1, b2, b3)
    gsems = (g0, g1, g2, g3)
    ssems = (s0, s1, s2, s3)

    dummy = zero_hbm.at[pl.ds(0, B)]

    def drain(sem, buf):
        pltpu.make_async_copy(dummy, buf, sem).wait()

    def zero_stripe():
        @pl.when(w < 15)
        def _():
            pltpu.sync_copy(zero_hbm,
                            acc.at[pl.ds(w * RPT_A, RPT_A)])

        @pl.when(w == 15)
        def _():
            pltpu.sync_copy(zero_hbm.at[pl.ds(0, RPT_B)],
                            acc.at[pl.ds(15 * RPT_A, RPT_B)])

    zero_stripe()
    plsc.subcore_barrier()

    for hop in range(4):
        zc = z_hbm.at[c] if hop == 0 else out_hbm.at[hop - 1].at[c]

        pltpu.sync_copy(src_hbm.at[w].at[pl.ds(0, CHW)], src_a)
        pltpu.sync_copy(dst_hbm.at[w].at[pl.ds(0, CHW)], dst_a)

        for ch in range(NCH):
            src_c, dst_c = (src_a, dst_a) if ch % 2 == 0 else (src_b, dst_b)
            src_n, dst_n = (src_b, dst_b) if ch % 2 == 0 else (src_a, dst_a)
            nxtw = CHW if ch + 1 < NCH - 1 else TB * B
            if ch + 1 < NCH:
                pltpu.async_copy(
                    src_hbm.at[w].at[pl.ds((ch + 1) * CHW, nxtw)],
                    src_n.at[pl.ds(0, nxtw)], si0)
                pltpu.async_copy(
                    dst_hbm.at[w].at[pl.ds((ch + 1) * CHW, nxtw)],
                    dst_n.at[pl.ds(0, nxtw)], si1)

            nb = CB if ch < NCH - 1 else TB

            if ch > 0:
                for u in range(4):
                    drain(ssems[u], bufs[u])
            for u in range(3):
                pltpu.async_copy(zc.at[src_c.at[pl.ds(u * B, B)]],
                                 bufs[u], gsems[u])

            def quad(k, carry):
                for u in range(4):
                    j = 4 * k + u
                    drain(gsems[u], bufs[u])
                    pltpu.async_copy(bufs[u],
                                     acc.at[dst_c.at[pl.ds(j * B, B)]],
                                     ssems[u], add=True)
                    v = (u + 3) % 4

                    @pl.when(j + 3 < nb)
                    def _():
                        @pl.when(j >= 1)
                        def _():
                            drain(ssems[v], bufs[v])
                        pltpu.async_copy(
                            zc.at[src_c.at[pl.ds((j + 3) * B, B)]],
                            bufs[v], gsems[v])
                return carry

            lax.fori_loop(0, nb // 4, quad, 0)

            if ch == NCH - 1:
                pltpu.sync_copy(tsrc_hbm.at[w], src_b.at[pl.ds(0, TSZ)])
                pltpu.sync_copy(tdst_hbm.at[w], dst_b.at[pl.ds(0, TSZ)])
                for t in range(2):
                    drain(ssems[t], bufs[t])
                    pltpu.async_copy(
                        zc.at[src_b.at[pl.ds(t * B, B)]], bufs[t], gsems[t])
                    drain(gsems[t], bufs[t])
                    pltpu.async_copy(bufs[t],
                                     acc.at[dst_b.at[pl.ds(t * B, B)]],
                                     ssems[t], add=True)

            if ch + 1 < NCH:
                pltpu.make_async_copy(
                    src_hbm.at[w].at[pl.ds((ch + 1) * CHW, nxtw)],
                    src_n.at[pl.ds(0, nxtw)], si0).wait()
                pltpu.make_async_copy(
                    dst_hbm.at[w].at[pl.ds((ch + 1) * CHW, nxtw)],
                    dst_n.at[pl.ds(0, nxtw)], si1).wait()

        for u in range(4):
            drain(ssems[u], bufs[u])

        plsc.subcore_barrier()

        @pl.when(w < 15)
        def _():
            pltpu.sync_copy(acc.at[pl.ds(w * RPT_A, RPT_A)],
                            out_hbm.at[hop].at[c].at[pl.ds(w * RPT_A, RPT_A)])

        @pl.when(w == 15)
        def _():
            pltpu.sync_copy(acc.at[pl.ds(15 * RPT_A, RPT_B)],
                            out_hbm.at[hop].at[c].at[pl.ds(15 * RPT_A, RPT_B)])

        if hop < 3:
            zero_stripe()
            plsc.subcore_barrier()


def _sc_hops(feat_a, edge_index):
    srcp = edge_index[0].reshape(NTILE, EPT)
    dstp = edge_index[1].reshape(NTILE, EPT)
    ntail = EPT - 4 * CHW - TB * B
    tsrc = jnp.concatenate(
        [srcp[:, EPT - ntail:],
         jnp.zeros((NTILE, TSZ - ntail), jnp.int32)], axis=1)
    tdst = jnp.concatenate(
        [dstp[:, EPT - ntail:],
         jnp.full((NTILE, TSZ - ntail), N, jnp.int32)], axis=1)
    zeros = jnp.zeros((RPT_A, H), jnp.float32)

    z = feat_a.reshape(N, 2, H).transpose(1, 0, 2)
    return _hops4(z, srcp, dstp, tsrc, tdst, zeros)


def _part_body(feat_a_ref, pm_ref, feat_b_ref, w_fuse_t_ref, w_prev_t_ref,
               bias_ref, part_ref, fw_ref):
    i = pl.program_id(0)

    @pl.when(i == 0)
    def _():
        fw_ref[...] = jnp.dot(feat_b_ref[...], w_fuse_t_ref[...],
                              preferred_element_type=jnp.float32)

    acc = jnp.dot(feat_a_ref[...], w_prev_t_ref[...],
                  preferred_element_type=jnp.float32)
    acc += jnp.dot(pm_ref[...], fw_ref[...],
                   preferred_element_type=jnp.float32)
    part_ref[...] = acc + bias_ref[...]


def _main_body(part_ref, z1l_ref, z1r_ref, z2l_ref, z2r_ref, z4l_ref,
               z4r_ref, w1l_ref, w1r_ref, w2l_ref, w2r_ref, w3l_ref,
               w3r_ref, r_ref, psum_ref, psq_ref):
    acc = part_ref[...]
    acc += jnp.dot(z1l_ref[0, 0], w1l_ref[...],
                   preferred_element_type=jnp.float32)
    acc += jnp.dot(z1r_ref[0, 0], w1r_ref[...],
                   preferred_element_type=jnp.float32)
    acc += jnp.dot(z2l_ref[0, 0], w2l_ref[...],
                   preferred_element_type=jnp.float32)
    acc += jnp.dot(z2r_ref[0, 0], w2r_ref[...],
                   preferred_element_type=jnp.float32)
    acc += jnp.dot(z4l_ref[0, 0], w3l_ref[...],
                   preferred_element_type=jnp.float32)
    acc += jnp.dot(z4r_ref[0, 0], w3r_ref[...],
                   preferred_element_type=jnp.float32)

    col = jax.lax.broadcasted_iota(jnp.int32, (BN, D), 1)
    acc = jnp.where(col >= D // 2, jnp.maximum(acc, 0.0), acc)

    r_ref[...] = acc
    psum_ref[...] = jnp.broadcast_to(jnp.sum(acc, axis=0, keepdims=True) / 8.0,
                                     (8, D))
    psq_ref[...] = jnp.broadcast_to(jnp.sum(acc * acc, axis=0, keepdims=True) / 8.0,
                                    (8, D))


def _bn_body(r_ref, psum_ref, psq_ref, gamma_ref, beta_ref, out_ref):
    mean = jnp.sum(psum_ref[...], axis=0, keepdims=True) / N
    var = jnp.sum(psq_ref[...], axis=0, keepdims=True) / N - mean * mean
    scale = jax.lax.rsqrt(var + 1e-5) * gamma_ref[...]
    out_ref[...] = (r_ref[...] - mean) * scale + beta_ref[...]


def _dense_part(zs, part, W_rad, bn_gamma, bn_beta):
    w1t = W_rad[0].T
    w2t = W_rad[1].T
    w3t = W_rad[2].T

    zspec = [
        pl.BlockSpec((1, 1, BN, H), lambda i, k=k, cc=cc: (k, cc, i, 0))
        for (k, cc) in ((0, 0), (0, 1), (1, 0), (1, 1), (3, 0), (3, 1))
    ]
    r, psum, psq = pl.pallas_call(
        _main_body,
        grid=(NBLK,),
        in_specs=[pl.BlockSpec((BN, D), lambda i: (i, 0))] + zspec + [
            pl.BlockSpec((H, D), lambda i: (0, 0)),
            pl.BlockSpec((H, D), lambda i: (1, 0)),
            pl.BlockSpec((H, D), lambda i: (0, 0)),
            pl.BlockSpec((H, D), lambda i: (1, 0)),
            pl.BlockSpec((H, D), lambda i: (0, 0)),
            pl.BlockSpec((H, D), lambda i: (1, 0)),
        ],
        out_specs=[
            pl.BlockSpec((BN, D), lambda i: (i, 0)),
            pl.BlockSpec((8, D), lambda i: (i, 0)),
            pl.BlockSpec((8, D), lambda i: (i, 0)),
        ],
        out_shape=[
            jax.ShapeDtypeStruct((N, D), jnp.float32),
            jax.ShapeDtypeStruct((NBLK * 8, D), jnp.float32),
            jax.ShapeDtypeStruct((NBLK * 8, D), jnp.float32),
        ],
    )(part, zs, zs, zs, zs, zs, zs, w1t, w1t, w2t, w2t, w3t, w3t)

    out = pl.pallas_call(
        _bn_body,
        grid=(NBLK,),
        in_specs=[
            pl.BlockSpec((BN, D), lambda i: (i, 0)),
            pl.BlockSpec((NBLK * 8, D), lambda i: (0, 0)),
            pl.BlockSpec((NBLK * 8, D), lambda i: (0, 0)),
            pl.BlockSpec((1, D), lambda i: (0, 0)),
            pl.BlockSpec((1, D), lambda i: (0, 0)),
        ],
        out_specs=pl.BlockSpec((BN, D), lambda i: (i, 0)),
        out_shape=jax.ShapeDtypeStruct((N, D), jnp.float32),
    )(r, psum, psq, bn_gamma.reshape(1, D), bn_beta.reshape(1, D))
    return out


def kernel(feat_a, feat_b, deg, pm_pd, edge_index,
           W_prev, b_prev, W_deg, b_deg, W_rad, b_rad,
           W_fuse, b_fuse, bn_gamma, bn_beta):
    bias = b_prev + b_rad[0] + b_rad[1] + b_rad[2] + b_fuse

    part = pl.pallas_call(
        _part_body,
        grid=(NBLK,),
        in_specs=[
            pl.BlockSpec((BN, D), lambda i: (i, 0)),
            pl.BlockSpec((BN, M), lambda i: (i, 0)),
            pl.BlockSpec((M, D), lambda i: (0, 0)),
            pl.BlockSpec((D, D), lambda i: (0, 0)),
            pl.BlockSpec((D, D), lambda i: (0, 0)),
            pl.BlockSpec((1, D), lambda i: (0, 0)),
        ],
        out_specs=pl.BlockSpec((BN, D), lambda i: (i, 0)),
        out_shape=jax.ShapeDtypeStruct((N, D), jnp.float32),
        scratch_shapes=[pltpu.VMEM((M, D), jnp.float32)],
    )(feat_a, pm_pd, feat_b, W_fuse.T, W_prev.T, bias.reshape(1, D))

    zs = _sc_hops(feat_a, edge_index)
    return _dense_part(zs, part, W_rad, bn_gamma, bn_beta)
